# Initial kernel scaffold; baseline (speedup 1.0000x reference)
import jax
import jax.numpy as jnp
from jax.experimental import pallas as pl


def kernel(input, embed):
    raise NotImplementedError("write your pallas kernel here")


# speedup vs baseline: 1.1823x; 1.1823x over previous
import functools

import jax
import jax.numpy as jnp
from jax import lax
from jax.experimental import pallas as pl
from jax.experimental.pallas import tpu as pltpu
from jax.experimental.pallas import tpu_sc as plsc

DIM = 256
TB = 512
CB = 2048


def _dist_argmin_body(n_tok, x_ref, e_ref, idx_ref, diff_ref, minv, mini, e2s):
    t = pl.program_id(0)
    c = pl.program_id(1)
    nt = pl.num_programs(0)
    nc = pl.num_programs(1)

    e = e_ref[...]

    @pl.when(t == 0)
    def _():
        e2s[c] = jnp.sum(e * e, axis=0, keepdims=True)

    x = x_ref[...]
    s = lax.dot_general(x.astype(jnp.bfloat16), e, (((1,), (0,)), ((), ())),
                        preferred_element_type=jnp.float32)
    x2 = jnp.sum(x * x, axis=1, keepdims=True)
    scores = (x2 - 2.0 * s) + e2s[c]
    lmin = jnp.min(scores, axis=1, keepdims=True)
    iota = lax.broadcasted_iota(jnp.int32, (TB, CB), 1) + c * CB
    larg = jnp.min(jnp.where(scores == lmin, iota, jnp.int32(2**30)),
                   axis=1, keepdims=True)

    prev_min = jnp.where(c == 0, jnp.inf, minv[...])
    prev_idx = jnp.where(c == 0, 0, mini[...])
    better = lmin < prev_min
    new_min = jnp.where(better, lmin, prev_min)
    new_idx = jnp.where(better, larg, prev_idx)
    minv[...] = new_min
    mini[...] = new_idx

    @pl.when(c == nc - 1)
    def _():
        idx_ref[...] = new_idx
        part = jnp.sum(new_min, keepdims=True)
        prev = jnp.where(t == 0, jnp.zeros((1, 1), jnp.float32), diff_ref[...])
        tot = prev + part
        diff_ref[...] = jnp.where(t == nt - 1, tot / (n_tok * DIM), tot)


def _transpose_body(e_ref, o_ref):
    o_ref[...] = e_ref[...].T


def _make_sc_gather(n_codes, n_tok):
    info = plsc.get_sparse_core_info()
    nw = info.num_cores * info.num_subcores
    bpw = n_tok // nw
    nch = bpw // 128
    mesh = plsc.VectorSubcoreMesh(core_axis_name="c", subcore_axis_name="s")

    @functools.partial(
        pl.kernel, mesh=mesh,
        out_type=jax.ShapeDtypeStruct((n_tok, DIM), jnp.float32),
        scratch_types=[
            pltpu.VMEM((nch, 128), jnp.int32),
            pltpu.VMEM((bpw, DIM), jnp.float32),
            pltpu.SemaphoreType.DMA,
        ],
    )
    def gather_k(table_hbm, idx_hbm, out_hbm, idx_v, rows_v, sem):
        wid = lax.axis_index("s") * info.num_cores + lax.axis_index("c")
        pltpu.sync_copy(idx_hbm.at[pl.ds(wid * nch, nch)], idx_v)
        cps = [pltpu.async_copy(table_hbm.at[idx_v.at[j]],
                                rows_v.at[pl.ds(j * 128, 128)], sem)
               for j in range(nch)]
        for cp in cps:
            cp.wait()
        pltpu.sync_copy(rows_v, out_hbm.at[pl.ds(wid * bpw, bpw)])

    return gather_k


def kernel(input, embed):
    x = input.reshape(-1, DIM)
    n_tok = x.shape[0]
    n_codes = embed.shape[1]
    nt, nc = n_tok // TB, n_codes // CB

    idx2, diff2 = pl.pallas_call(
        functools.partial(_dist_argmin_body, n_tok),
        grid=(nt, nc),
        in_specs=[
            pl.BlockSpec((TB, DIM), lambda t, c: (t, 0)),
            pl.BlockSpec((DIM, CB), lambda t, c: (0, c)),
        ],
        out_specs=[
            pl.BlockSpec((TB, 1), lambda t, c: (t, 0)),
            pl.BlockSpec((1, 1), lambda t, c: (0, 0)),
        ],
        out_shape=[
            jax.ShapeDtypeStruct((n_tok, 1), jnp.int32),
            jax.ShapeDtypeStruct((1, 1), jnp.float32),
        ],
        scratch_shapes=[
            pltpu.VMEM((TB, 1), jnp.float32),
            pltpu.VMEM((TB, 1), jnp.int32),
            pltpu.VMEM((nc, 1, CB), jnp.float32),
        ],
    )(x, embed)

    table = pl.pallas_call(
        _transpose_body,
        grid=(nc,),
        in_specs=[pl.BlockSpec((DIM, CB), lambda c: (0, c))],
        out_specs=pl.BlockSpec((CB, DIM), lambda c: (c, 0)),
        out_shape=jax.ShapeDtypeStruct((n_codes, DIM), jnp.float32),
    )(embed)

    idx = idx2.reshape(-1)
    q = _make_sc_gather(n_codes, n_tok)(table, idx.reshape(-1, 128))
    quantize = q.reshape(input.shape)
    diff = diff2[0, 0]
    embed_ind = idx.reshape(input.shape[:-1])
    return (quantize, diff, embed_ind)


# speedup vs baseline: 1.1855x; 1.0027x over previous
import functools

import jax
import jax.numpy as jnp
from jax import lax
from jax.experimental import pallas as pl
from jax.experimental.pallas import tpu as pltpu
from jax.experimental.pallas import tpu_sc as plsc

DIM = 256
TB = 512
CB = 2048


def _dist_argmin_body(n_tok, x_ref, e_ref, idx_ref, diff_ref, minv, mini, e2s):
    t = pl.program_id(0)
    c = pl.program_id(1)
    nt = pl.num_programs(0)
    nc = pl.num_programs(1)

    e = e_ref[...]

    @pl.when(t == 0)
    def _():
        e2s[c] = jnp.sum(e * e, axis=0, keepdims=True)

    x = x_ref[...]
    s = lax.dot_general(x.astype(jnp.bfloat16), e.astype(jnp.bfloat16),
                        (((1,), (0,)), ((), ())),
                        preferred_element_type=jnp.float32)
    x2 = jnp.sum(x * x, axis=1, keepdims=True)
    scores = (x2 - 2.0 * s) + e2s[c]
    lmin = jnp.min(scores, axis=1, keepdims=True)
    iota = lax.broadcasted_iota(jnp.int32, (TB, CB), 1) + c * CB
    larg = jnp.min(jnp.where(scores == lmin, iota, jnp.int32(2**30)),
                   axis=1, keepdims=True)

    prev_min = jnp.where(c == 0, jnp.inf, minv[...])
    prev_idx = jnp.where(c == 0, 0, mini[...])
    better = lmin < prev_min
    new_min = jnp.where(better, lmin, prev_min)
    new_idx = jnp.where(better, larg, prev_idx)
    minv[...] = new_min
    mini[...] = new_idx

    @pl.when(c == nc - 1)
    def _():
        idx_ref[...] = new_idx
        part = jnp.sum(new_min, keepdims=True)
        prev = jnp.where(t == 0, jnp.zeros((1, 1), jnp.float32), diff_ref[...])
        tot = prev + part
        diff_ref[...] = jnp.where(t == nt - 1, tot / (n_tok * DIM), tot)


def _transpose_body(e_ref, o_ref):
    o_ref[...] = e_ref[...].T


def _make_sc_gather(n_codes, n_tok):
    info = plsc.get_sparse_core_info()
    nw = info.num_cores * info.num_subcores
    bpw = n_tok // nw
    nch = bpw // 128
    mesh = plsc.VectorSubcoreMesh(core_axis_name="c", subcore_axis_name="s")

    @functools.partial(
        pl.kernel, mesh=mesh,
        out_type=jax.ShapeDtypeStruct((n_tok, DIM), jnp.float32),
        scratch_types=[
            pltpu.VMEM((nch, 128), jnp.int32),
            pltpu.VMEM((bpw, DIM), jnp.float32),
            pltpu.SemaphoreType.DMA,
        ],
    )
    def gather_k(table_hbm, idx_hbm, out_hbm, idx_v, rows_v, sem):
        wid = lax.axis_index("s") * info.num_cores + lax.axis_index("c")
        pltpu.sync_copy(idx_hbm.at[pl.ds(wid * nch, nch)], idx_v)
        cps = [pltpu.async_copy(table_hbm.at[idx_v.at[j]],
                                rows_v.at[pl.ds(j * 128, 128)], sem)
               for j in range(nch)]
        for cp in cps:
            cp.wait()
        pltpu.sync_copy(rows_v, out_hbm.at[pl.ds(wid * bpw, bpw)])

    return gather_k


def kernel(input, embed):
    x = input.reshape(-1, DIM)
    n_tok = x.shape[0]
    n_codes = embed.shape[1]
    nt, nc = n_tok // TB, n_codes // CB

    idx2, diff2 = pl.pallas_call(
        functools.partial(_dist_argmin_body, n_tok),
        grid=(nt, nc),
        in_specs=[
            pl.BlockSpec((TB, DIM), lambda t, c: (t, 0)),
            pl.BlockSpec((DIM, CB), lambda t, c: (0, c)),
        ],
        out_specs=[
            pl.BlockSpec((TB, 1), lambda t, c: (t, 0)),
            pl.BlockSpec((1, 1), lambda t, c: (0, 0)),
        ],
        out_shape=[
            jax.ShapeDtypeStruct((n_tok, 1), jnp.int32),
            jax.ShapeDtypeStruct((1, 1), jnp.float32),
        ],
        scratch_shapes=[
            pltpu.VMEM((TB, 1), jnp.float32),
            pltpu.VMEM((TB, 1), jnp.int32),
            pltpu.VMEM((nc, 1, CB), jnp.float32),
        ],
    )(x, embed)

    table = pl.pallas_call(
        _transpose_body,
        grid=(nc,),
        in_specs=[pl.BlockSpec((DIM, CB), lambda c: (0, c))],
        out_specs=pl.BlockSpec((CB, DIM), lambda c: (c, 0)),
        out_shape=jax.ShapeDtypeStruct((n_codes, DIM), jnp.float32),
    )(embed)

    idx = idx2.reshape(-1)
    q = _make_sc_gather(n_codes, n_tok)(table, idx.reshape(-1, 128))
    quantize = q.reshape(input.shape)
    diff = diff2[0, 0]
    embed_ind = idx.reshape(input.shape[:-1])
    return (quantize, diff, embed_ind)
